# Initial kernel scaffold; baseline (speedup 1.0000x reference)
import jax
import jax.numpy as jnp
from jax.experimental import pallas as pl


def kernel(mm_embedding):
    raise NotImplementedError("write your pallas kernel here")


# speedup vs baseline: 9.5321x; 9.5321x over previous
import functools

import jax
import jax.numpy as jnp
from jax.experimental import pallas as pl
from jax.experimental.pallas import tpu as pltpu

_K = 10
_N = 8192
_D = 256
_BR = 256
_NBLK = _N // _BR


def _knn_kernel(x_ref, idx_ref, val_ref, xn_ref):
    i = pl.program_id(0)

    @pl.when(i == 0)
    def _normalize():
        x = x_ref[...]
        n2 = jnp.sum(x * x, axis=1, keepdims=True)
        xn_ref[...] = x / jnp.sqrt(n2)

    xn_blk = xn_ref[pl.ds(i * _BR, _BR), :]
    s = jax.lax.dot_general(
        xn_blk, xn_ref[...], (((1,), (1,)), ((), ())),
        preferred_element_type=jnp.float32)

    col_ids = jax.lax.broadcasted_iota(jnp.int32, (_BR, _N), 1)
    picked = []
    for _ in range(_K):
        m = jnp.max(s, axis=1, keepdims=True)
        idx = jnp.min(jnp.where(s == m, col_ids, _N), axis=1, keepdims=True)
        picked.append(idx)
        s = jnp.where(col_ids == idx, -jnp.inf, s)
    idx_ref[...] = jnp.concatenate(picked, axis=1)

    row_sum = jnp.float32(1e-7) + jnp.float32(_K)
    r_inv_sqrt = row_sum ** -0.5
    val_ref[...] = jnp.full((_BR, _K), r_inv_sqrt * r_inv_sqrt, jnp.float32)


@functools.partial(jax.jit)
def kernel(mm_embedding):
    knn_ind, vals = pl.pallas_call(
        _knn_kernel,
        grid=(_NBLK,),
        in_specs=[pl.BlockSpec((_N, _D), lambda i: (0, 0))],
        out_specs=[
            pl.BlockSpec((_BR, _K), lambda i: (i, 0)),
            pl.BlockSpec((_BR, _K), lambda i: (i, 0)),
        ],
        out_shape=[
            jax.ShapeDtypeStruct((_N, _K), jnp.int32),
            jax.ShapeDtypeStruct((_N, _K), jnp.float32),
        ],
        scratch_shapes=[pltpu.VMEM((_N, _D), jnp.float32)],
    )(mm_embedding)

    rows = jnp.broadcast_to(jnp.arange(_N)[:, None], (_N, _K)).reshape(-1)
    indices = jnp.stack((rows, knn_ind.reshape(-1)), axis=0)
    return (indices, vals.reshape(-1))
